# Initial kernel scaffold; baseline (speedup 1.0000x reference)
import jax
import jax.numpy as jnp
from jax.experimental import pallas as pl


def kernel(x, codebooks):
    raise NotImplementedError("write your pallas kernel here")


# speedup vs baseline: 1.0997x; 1.0997x over previous
import functools

import jax
import jax.numpy as jnp
from jax import lax
from jax.experimental import pallas as pl
from jax.experimental.pallas import tpu as pltpu
from jax.experimental.pallas import tpu_sc as plsc

B, H, W, D = 4, 32, 32, 256
DEPTH = 4
K = 8192
M = B * H * W
TM = 256

NC, NS = 2, 16
NWORK = NC * NS
TPW = M // NWORK


def _argmin_body(r_ref, cb_ref, idx_ref):
    r = r_ref[...]
    cb = cb_ref[...]
    mm = lax.dot_general(r, cb, (((1,), (1,)), ((), ())),
                         preferred_element_type=jnp.float32)
    in_sq = jnp.sum(r * r, axis=1, keepdims=True)
    cb_sq = jnp.sum(cb * cb, axis=1)[None, :]
    dist = in_sq + cb_sq - 2.0 * mm
    best = jnp.min(dist, axis=1, keepdims=True)
    ids = lax.broadcasted_iota(jnp.int32, dist.shape, 1)
    idx_ref[...] = jnp.min(jnp.where(dist == best, ids, K), axis=1)


def _tc_argmin(r_flat, cb):
    return pl.pallas_call(
        _argmin_body,
        grid=(M // TM,),
        in_specs=[
            pl.BlockSpec((TM, D), lambda m: (m, 0)),
            pl.BlockSpec((K, D), lambda m: (0, 0)),
        ],
        out_specs=pl.BlockSpec((TM,), lambda m: (m,)),
        out_shape=jax.ShapeDtypeStruct((M,), jnp.int32),
    )(r_flat, cb)


@functools.cache
def _make_sc_update():
    mesh = plsc.VectorSubcoreMesh(core_axis_name="c", subcore_axis_name="s")

    @functools.partial(
        pl.kernel,
        mesh=mesh,
        out_type=[
            jax.ShapeDtypeStruct((M, D), jnp.float32),
            jax.ShapeDtypeStruct((NWORK, 16), jnp.float32),
        ],
        scratch_types=[
            pltpu.VMEM((TPW,), jnp.int32),
            pltpu.VMEM((TPW, D), jnp.float32),
            pltpu.VMEM((TPW, D), jnp.float32),
            pltpu.VMEM((16,), jnp.float32),
            pltpu.SemaphoreType.DMA,
        ],
    )
    def sc_update(cb_hbm, idx_hbm, r_hbm, rout_hbm, ss_hbm,
                  idx_v, q_v, r_v, acc_v, sem):
        wid = lax.axis_index("s") * NC + lax.axis_index("c")
        base = wid * TPW
        pltpu.sync_copy(idx_hbm.at[pl.ds(base, TPW)], idx_v)
        gather = pltpu.async_copy(cb_hbm.at[idx_v], q_v, sem)
        pltpu.sync_copy(r_hbm.at[pl.ds(base, TPW), :], r_v)
        gather.wait()

        def token_body(t, acc):
            for c in range(D // 16):
                sl = pl.ds(c * 16, 16)
                v = r_v[t, sl] - q_v[t, sl]
                r_v[t, sl] = v
                acc = acc + v * v
            return acc

        acc = lax.fori_loop(0, TPW, token_body, jnp.zeros((16,), jnp.float32))
        acc_v[...] = acc
        pltpu.sync_copy(r_v, rout_hbm.at[pl.ds(base, TPW), :])
        pltpu.sync_copy(acc_v, ss_hbm.at[wid])

    return sc_update


def kernel(x, codebooks):
    r = x.reshape(M, D)
    codes = []
    losses = []
    for i in range(DEPTH):
        idx = _tc_argmin(r, codebooks[i])
        r, ss = _make_sc_update()(codebooks[i], idx, r)
        codes.append(idx)
        losses.append(jnp.sum(ss) / (M * D))
    quants_trunc = x - r.reshape(x.shape)
    commitment_loss = jnp.mean(jnp.stack(losses))
    codes_arr = jnp.stack(codes, axis=-1).reshape(B, H, W, DEPTH)
    return quants_trunc, commitment_loss, codes_arr


# speedup vs baseline: 1.2643x; 1.1497x over previous
import functools

import jax
import jax.numpy as jnp
from jax import lax
from jax.experimental import pallas as pl
from jax.experimental.pallas import tpu as pltpu
from jax.experimental.pallas import tpu_sc as plsc

B, H, W, D = 4, 32, 32, 256
DEPTH = 4
K = 8192
M = B * H * W
TM = 1024
CK = 512

NC, NS = 2, 16
NWORK = NC * NS
TPW = M // NWORK


def _argmin_body(r_ref, cb_ref, idx_ref):
    r = r_ref[...]
    r2 = r + r
    in_sq = jnp.sum(r * r, axis=1, keepdims=True)
    lane = lax.broadcasted_iota(jnp.int32, (TM, 128), 1)

    def chunk(c, carry):
        rm, rmi = carry
        cbc = cb_ref[pl.ds(c * CK, CK), :]
        mm2 = lax.dot_general(r2, cbc, (((1,), (1,)), ((), ())),
                              preferred_element_type=jnp.float32)
        cb_sq = jnp.sum(cbc * cbc, axis=1)[None, :]
        d = (in_sq + cb_sq) - mm2
        for j in range(CK // 128):
            dj = d[:, j * 128:(j + 1) * 128]
            idj = lane + (c * CK + j * 128)
            better = dj < rm
            rm = jnp.where(better, dj, rm)
            rmi = jnp.where(better, idj, rmi)
        return rm, rmi

    rm0 = jnp.full((TM, 128), jnp.inf, jnp.float32)
    rmi0 = jnp.full((TM, 128), K, jnp.int32)
    rm, rmi = lax.fori_loop(0, K // CK, chunk, (rm0, rmi0))
    best = jnp.min(rm, axis=1, keepdims=True)
    idx_ref[...] = jnp.min(jnp.where(rm == best, rmi, K), axis=1)


def _tc_argmin(r_flat, cb):
    return pl.pallas_call(
        _argmin_body,
        grid=(M // TM,),
        in_specs=[
            pl.BlockSpec((TM, D), lambda m: (m, 0)),
            pl.BlockSpec((K, D), lambda m: (0, 0)),
        ],
        out_specs=pl.BlockSpec((TM,), lambda m: (m,)),
        out_shape=jax.ShapeDtypeStruct((M,), jnp.int32),
    )(r_flat, cb)


@functools.cache
def _make_sc_update():
    mesh = plsc.VectorSubcoreMesh(core_axis_name="c", subcore_axis_name="s")

    @functools.partial(
        pl.kernel,
        mesh=mesh,
        out_type=[
            jax.ShapeDtypeStruct((M, D), jnp.float32),
            jax.ShapeDtypeStruct((NWORK, 16), jnp.float32),
        ],
        scratch_types=[
            pltpu.VMEM((TPW,), jnp.int32),
            pltpu.VMEM((TPW, D), jnp.float32),
            pltpu.VMEM((TPW, D), jnp.float32),
            pltpu.VMEM((16,), jnp.float32),
            pltpu.SemaphoreType.DMA,
        ],
    )
    def sc_update(cb_hbm, idx_hbm, r_hbm, rout_hbm, ss_hbm,
                  idx_v, q_v, r_v, acc_v, sem):
        wid = lax.axis_index("s") * NC + lax.axis_index("c")
        base = wid * TPW
        pltpu.sync_copy(idx_hbm.at[pl.ds(base, TPW)], idx_v)
        gather = pltpu.async_copy(cb_hbm.at[idx_v], q_v, sem)
        pltpu.sync_copy(r_hbm.at[pl.ds(base, TPW), :], r_v)
        gather.wait()

        def token_body(t, acc):
            for c in range(D // 16):
                sl = pl.ds(c * 16, 16)
                v = r_v[t, sl] - q_v[t, sl]
                r_v[t, sl] = v
                acc = acc + v * v
            return acc

        acc = lax.fori_loop(0, TPW, token_body, jnp.zeros((16,), jnp.float32))
        acc_v[...] = acc
        pltpu.sync_copy(r_v, rout_hbm.at[pl.ds(base, TPW), :])
        pltpu.sync_copy(acc_v, ss_hbm.at[wid])

    return sc_update


def kernel(x, codebooks):
    r = x.reshape(M, D)
    codes = []
    losses = []
    for i in range(DEPTH):
        idx = _tc_argmin(r, codebooks[i])
        r, ss = _make_sc_update()(codebooks[i], idx, r)
        codes.append(idx)
        losses.append(jnp.sum(ss) / (M * D))
    quants_trunc = x - r.reshape(x.shape)
    commitment_loss = jnp.mean(jnp.stack(losses))
    codes_arr = jnp.stack(codes, axis=-1).reshape(B, H, W, DEPTH)
    return quants_trunc, commitment_loss, codes_arr
